# baseline (speedup 1.0000x reference)
import functools

import jax
import jax.numpy as jnp
from jax import lax
from jax.experimental import pallas as pl
from jax.experimental.pallas import tpu as pltpu
from jax.experimental.pallas import tpu_sc as plsc

_NC = 2
_NS = 16
_NW = _NC * _NS
_CH = 128

_SELU_ALPHA = 1.6732632423543772
_SELU_SCALE = 1.0507009873554805


def _selu(x):
    return _SELU_SCALE * jnp.where(x > 0, x, _SELU_ALPHA * (jnp.exp(x) - 1.0))


def _sc_deg(dst2d, ew2d, NP):
    EPr = dst2d.shape[0]
    rpt = EPr // _NW
    ZB = NP // _NS
    mesh = plsc.VectorSubcoreMesh(core_axis_name="c", subcore_axis_name="s")

    def body(dst_hbm, ew_hbm, out_hbm, dst_v, ew_v, deg_sh, zb, isem, ssem):
        cid = lax.axis_index("c")
        sid = lax.axis_index("s")
        wid = sid * _NC + cid
        row0 = wid * rpt

        d1 = pltpu.async_copy(dst_hbm.at[pl.ds(row0, rpt)], dst_v, isem)
        d2 = pltpu.async_copy(ew_hbm.at[pl.ds(row0, rpt)], ew_v, isem)

        def zf(i, carry):
            zb[pl.ds(i * 16, 16)] = jnp.zeros((16,), jnp.float32)
            return carry

        lax.fori_loop(0, ZB // 16, zf, 0)
        pltpu.sync_copy(zb, deg_sh.at[pl.ds(sid * ZB, ZB)])
        d1.wait()
        d2.wait()
        plsc.subcore_barrier()

        def fire(k, carry):
            pltpu.async_copy(ew_v.at[k], deg_sh.at[dst_v.at[k]], ssem,
                             add=True)
            return carry

        lax.fori_loop(0, rpt, fire, 0)

        def drain(k, carry):
            pltpu.make_async_copy(ew_v.at[0], deg_sh.at[dst_v.at[0]],
                                  ssem).wait()
            return carry

        lax.fori_loop(0, rpt, drain, 0)
        plsc.subcore_barrier()
        pltpu.sync_copy(deg_sh.at[pl.ds(sid * ZB, ZB)],
                        out_hbm.at[pl.ds(cid * NP + sid * ZB, ZB)])

    return pl.kernel(
        body,
        out_type=jax.ShapeDtypeStruct((_NC * NP,), jnp.float32),
        mesh=mesh,
        scratch_types=[
            pltpu.VMEM((rpt, _CH), jnp.int32),
            pltpu.VMEM((rpt, _CH), jnp.float32),
            pltpu.VMEM_SHARED((NP,), jnp.float32),
            pltpu.VMEM((ZB,), jnp.float32),
            pltpu.SemaphoreType.DMA,
            pltpu.SemaphoreType.DMA,
        ],
    )(dst2d, ew2d)


def _sc_msg(src2d, dst2d, ew2d, xs, NP, D):
    EPr = src2d.shape[0]
    rpt = EPr // _NW
    ZB = NP // _NS
    ZBR = ZB // 4
    mesh = plsc.VectorSubcoreMesh(core_axis_name="c", subcore_axis_name="s")

    def body(src_hbm, dst_hbm, ew_hbm, xs_hbm, out_hbm,
             src_v, dst_v, ew_v, rows0, rows1, rows2, rows3, out_sh, zb,
             gsem0, gsem1, gsem2, gsem3, ssem0, ssem1, ssem2, ssem3, isem):
        cid = lax.axis_index("c")
        sid = lax.axis_index("s")
        wid = sid * _NC + cid
        row0 = wid * rpt
        rows = (rows0, rows1, rows2, rows3)
        gsem = (gsem0, gsem1, gsem2, gsem3)
        ssem = (ssem0, ssem1, ssem2, ssem3)

        d1 = pltpu.async_copy(src_hbm.at[pl.ds(row0, rpt)], src_v, isem)
        d2 = pltpu.async_copy(dst_hbm.at[pl.ds(row0, rpt)], dst_v, isem)
        d3 = pltpu.async_copy(ew_hbm.at[pl.ds(row0, rpt)], ew_v, isem)

        def zf(i, carry):
            for g in range(D // 16):
                zb[i, pl.ds(g * 16, 16)] = jnp.zeros((16,), jnp.float32)
            return carry

        lax.fori_loop(0, ZBR, zf, 0)
        for t in range(4):
            pltpu.sync_copy(zb, out_sh.at[pl.ds(sid * ZB + t * ZBR, ZBR)])
        d1.wait()
        d2.wait()
        d3.wait()
        plsc.subcore_barrier()

        def g_start(k, b):
            pltpu.async_copy(xs_hbm.at[src_v.at[k]], rows[b], gsem[b])

        def g_wait(b):
            pltpu.make_async_copy(xs_hbm.at[src_v.at[0]], rows[b],
                                  gsem[b]).wait()

        def s_start(k, b):
            pltpu.async_copy(rows[b], out_sh.at[dst_v.at[k]], ssem[b],
                             add=True)

        def s_wait(k, b):
            pltpu.make_async_copy(rows[b], out_sh.at[dst_v.at[k]],
                                  ssem[b]).wait()

        def scale(k, b):
            def grp(g, c2):
                base = g * 16
                wv = ew_v[k, pl.ds(base, 16)]
                for l in range(16):
                    bv = jnp.full((16,), wv[l], jnp.float32)
                    e = base + l
                    for q in range(D // 16):
                        sl = pl.ds(q * 16, 16)
                        rows[b][e, sl] = rows[b][e, sl] * bv
                return c2

            lax.fori_loop(0, _CH // 16, grp, 0, unroll=4)

        for b in range(4):
            g_start(b, b)

        def quad(i, carry):
            for b in range(4):
                k = 4 * i + b
                g_wait(b)
                scale(k, b)
                s_start(k, b)
            for b in range(4):
                k = 4 * i + b
                s_wait(k, b)

                @pl.when(k + 4 < rpt)
                def _():
                    g_start(k + 4, b)
            return carry

        lax.fori_loop(0, rpt // 4, quad, 0)
        plsc.subcore_barrier()
        pltpu.sync_copy(out_sh.at[pl.ds(sid * ZB, ZB)],
                        out_hbm.at[cid, pl.ds(sid * ZB, ZB)])

    return pl.kernel(
        body,
        out_type=jax.ShapeDtypeStruct((_NC, NP, D), jnp.float32),
        mesh=mesh,
        scratch_types=[
            pltpu.VMEM((rpt, _CH), jnp.int32),
            pltpu.VMEM((rpt, _CH), jnp.int32),
            pltpu.VMEM((rpt, _CH), jnp.float32),
            pltpu.VMEM((_CH, D), jnp.float32),
            pltpu.VMEM((_CH, D), jnp.float32),
            pltpu.VMEM((_CH, D), jnp.float32),
            pltpu.VMEM((_CH, D), jnp.float32),
            pltpu.VMEM_SHARED((NP, D), jnp.float32),
            pltpu.VMEM((ZBR, D), jnp.float32),
            pltpu.SemaphoreType.DMA,
            pltpu.SemaphoreType.DMA,
            pltpu.SemaphoreType.DMA,
            pltpu.SemaphoreType.DMA,
            pltpu.SemaphoreType.DMA,
            pltpu.SemaphoreType.DMA,
            pltpu.SemaphoreType.DMA,
            pltpu.SemaphoreType.DMA,
            pltpu.SemaphoreType.DMA,
        ],
        compiler_params=pltpu.CompilerParams(use_tc_tiling_on_sc=False),
    )(src2d, dst2d, ew2d, xs)


def _tc_prep(deg_p, xpad, Wc1):
    NPl, D = xpad.shape

    def body(degp_ref, x_ref, w_ref, dinv_ref, xs_ref):
        deg = degp_ref[0:1, :] + degp_ref[1:2, :] + 1.0
        dinv = lax.rsqrt(deg)
        dcol = jnp.transpose(dinv)
        xw = jnp.dot(x_ref[...], w_ref[...], preferred_element_type=jnp.float32)
        dinv_ref[...] = dcol
        xs_ref[...] = dcol * xw

    return pl.pallas_call(
        body,
        out_shape=[jax.ShapeDtypeStruct((NPl, 1), jnp.float32),
                   jax.ShapeDtypeStruct((NPl, D), jnp.float32)],
    )(deg_p, xpad, Wc1)


def _tc_mid(S_p, xs, dinv, bc, Wc):
    NPl, D = xs.shape

    def body(s_ref, xs_ref, dinv_ref, bc_ref, w_ref, out_ref):
        pre = dinv_ref[...] * (s_ref[0] + s_ref[1] + xs_ref[...]) + bc_ref[...]
        h = _selu(pre)
        out_ref[...] = dinv_ref[...] * jnp.dot(
            h, w_ref[...], preferred_element_type=jnp.float32)

    return pl.pallas_call(
        body,
        out_shape=jax.ShapeDtypeStruct((NPl, D), jnp.float32),
    )(S_p, xs, dinv, bc, Wc)


def _tc_final(S_p, xs2, dinv, bc2, W1t, b1t, W2t, b2t, g_col, lb_col,
              block=640):
    D = b1t.shape[0]
    N = b1t.shape[1]
    NPl = xs2.shape[0]
    grid = NPl // block

    def body(s_ref, xs_ref, dinv_ref, bc_ref, w1_ref, b1_ref,
             w2_ref, b2_ref, g_ref, lb_ref, out_ref):
        pre = dinv_ref[...] * (s_ref[0] + s_ref[1] + xs_ref[...]) + bc_ref[...]
        h = _selu(pre)
        ht = jnp.transpose(h)
        t = jnp.sum(w1_ref[...] * ht[:, None, :], axis=0) + b1_ref[...]
        mu = jnp.mean(t, axis=0, keepdims=True)
        var = jnp.mean((t - mu) ** 2, axis=0, keepdims=True)
        t = (t - mu) * lax.rsqrt(var + 1e-5) * g_ref[...] + lb_ref[...]
        out_ref[...] = jnp.sum(w2_ref[...] * t[:, None, :], axis=0) + b2_ref[...]

    return pl.pallas_call(
        body,
        grid=(grid,),
        in_specs=[
            pl.BlockSpec((2, block, D), lambda i: (0, i, 0)),
            pl.BlockSpec((block, D), lambda i: (i, 0)),
            pl.BlockSpec((block, 1), lambda i: (i, 0)),
            pl.BlockSpec((1, D), lambda i: (0, 0)),
            pl.BlockSpec((D, D, block), lambda i: (0, 0, i)),
            pl.BlockSpec((D, block), lambda i: (0, i)),
            pl.BlockSpec((D, D, block), lambda i: (0, 0, i)),
            pl.BlockSpec((D, block), lambda i: (0, i)),
            pl.BlockSpec((D, 1), lambda i: (0, 0)),
            pl.BlockSpec((D, 1), lambda i: (0, 0)),
        ],
        out_specs=pl.BlockSpec((D, block), lambda i: (0, i)),
        out_shape=jax.ShapeDtypeStruct((D, N), jnp.float32),
    )(S_p, xs2, dinv, bc2, W1t, b1t, W2t, b2t, g_col, lb_col)


def kernel(x, edge_index, edge_weight, Wc1, bc1, Wc2, bc2, W1, b1, W2, b2,
           ln_g, ln_b):
    N, D = x.shape
    E = edge_weight.shape[0]
    NP = ((N + 2047) // 2048) * 2048
    if NP == N:
        NP = N + 2048
    EP = ((E + _NW * _CH - 1) // (_NW * _CH)) * (_NW * _CH)
    pad = EP - E

    src = edge_index[0].astype(jnp.int32)
    dst = edge_index[1].astype(jnp.int32)
    ar = jnp.arange(pad, dtype=jnp.int32)
    src_p = jnp.concatenate([src, ar % N]).reshape(EP // _CH, _CH)
    dst_p = jnp.concatenate([dst, N + ar % (NP - N)]).reshape(EP // _CH, _CH)
    ew_p = jnp.concatenate(
        [edge_weight, jnp.zeros((pad,), edge_weight.dtype)]).reshape(EP // _CH, _CH)
    xpad = jnp.concatenate([x, jnp.zeros((NP - N, D), x.dtype)], axis=0)

    deg_p = _sc_deg(dst_p, ew_p, NP).reshape(_NC, NP)
    dinv, xs1 = _tc_prep(deg_p, xpad, Wc1)
    S1 = _sc_msg(src_p, dst_p, ew_p, xs1, NP, D)
    xs2 = _tc_mid(S1, xs1, dinv, bc1.reshape(1, D), Wc2)
    S2 = _sc_msg(src_p, dst_p, ew_p, xs2, NP, D)
    W1t = jnp.transpose(W1, (1, 2, 0))
    W2t = jnp.transpose(W2, (1, 2, 0))
    out_t = _tc_final(S2, xs2, dinv, bc2.reshape(1, D), W1t, jnp.transpose(b1),
                      W2t, jnp.transpose(b2), ln_g.reshape(D, 1),
                      ln_b.reshape(D, 1))
    return jnp.transpose(out_t)


# speedup vs baseline: 1.0091x; 1.0091x over previous
import functools

import jax
import jax.numpy as jnp
from jax import lax
from jax.experimental import pallas as pl
from jax.experimental.pallas import tpu as pltpu
from jax.experimental.pallas import tpu_sc as plsc

_NC = 2
_NS = 16
_NW = _NC * _NS
_CH = 128

_SELU_ALPHA = 1.6732632423543772
_SELU_SCALE = 1.0507009873554805


def _selu(x):
    return _SELU_SCALE * jnp.where(x > 0, x, _SELU_ALPHA * (jnp.exp(x) - 1.0))


def _sc_deg(dst2d, ew2d, NP):
    EPr = dst2d.shape[0]
    rpt = EPr // _NW
    ZB = NP // _NS
    mesh = plsc.VectorSubcoreMesh(core_axis_name="c", subcore_axis_name="s")

    def body(dst_hbm, ew_hbm, out_hbm, dst_v, ew_v, deg_sh, zb, isem, ssem):
        cid = lax.axis_index("c")
        sid = lax.axis_index("s")
        wid = sid * _NC + cid
        row0 = wid * rpt

        d1 = pltpu.async_copy(dst_hbm.at[pl.ds(row0, rpt)], dst_v, isem)
        d2 = pltpu.async_copy(ew_hbm.at[pl.ds(row0, rpt)], ew_v, isem)

        def zf(i, carry):
            zb[pl.ds(i * 16, 16)] = jnp.zeros((16,), jnp.float32)
            return carry

        lax.fori_loop(0, ZB // 16, zf, 0)
        pltpu.sync_copy(zb, deg_sh.at[pl.ds(sid * ZB, ZB)])
        d1.wait()
        d2.wait()
        plsc.subcore_barrier()

        def fire(k, carry):
            pltpu.async_copy(ew_v.at[k], deg_sh.at[dst_v.at[k]], ssem,
                             add=True)
            return carry

        lax.fori_loop(0, rpt, fire, 0)

        def drain(k, carry):
            pltpu.make_async_copy(ew_v.at[0], deg_sh.at[dst_v.at[0]],
                                  ssem).wait()
            return carry

        lax.fori_loop(0, rpt, drain, 0)
        plsc.subcore_barrier()
        pltpu.sync_copy(deg_sh.at[pl.ds(sid * ZB, ZB)],
                        out_hbm.at[pl.ds(cid * NP + sid * ZB, ZB)])

    return pl.kernel(
        body,
        out_type=jax.ShapeDtypeStruct((_NC * NP,), jnp.float32),
        mesh=mesh,
        scratch_types=[
            pltpu.VMEM((rpt, _CH), jnp.int32),
            pltpu.VMEM((rpt, _CH), jnp.float32),
            pltpu.VMEM_SHARED((NP,), jnp.float32),
            pltpu.VMEM((ZB,), jnp.float32),
            pltpu.SemaphoreType.DMA,
            pltpu.SemaphoreType.DMA,
        ],
    )(dst2d, ew2d)


def _sc_msg(src2d, dst2d, ew2d, xs, NP, D):
    EPr = src2d.shape[0]
    rpt = EPr // _NW
    ZB = NP // _NS
    ZBR = ZB // 4
    mesh = plsc.VectorSubcoreMesh(core_axis_name="c", subcore_axis_name="s")

    def body(src_hbm, dst_hbm, ew_hbm, xs_hbm, out_hbm,
             src_v, dst_v, ew_v, rows0, rows1, rows2, rows3, out_sh, zb,
             gsem0, gsem1, gsem2, gsem3, ssem0, ssem1, ssem2, ssem3, isem):
        cid = lax.axis_index("c")
        sid = lax.axis_index("s")
        wid = sid * _NC + cid
        row0 = wid * rpt
        rows = (rows0, rows1, rows2, rows3)
        gsem = (gsem0, gsem1, gsem2, gsem3)
        ssem = (ssem0, ssem1, ssem2, ssem3)

        d1 = pltpu.async_copy(src_hbm.at[pl.ds(row0, rpt)], src_v, isem)
        d2 = pltpu.async_copy(dst_hbm.at[pl.ds(row0, rpt)], dst_v, isem)
        d3 = pltpu.async_copy(ew_hbm.at[pl.ds(row0, rpt)], ew_v, isem)

        def zf(i, carry):
            for g in range(D // 16):
                zb[i, pl.ds(g * 16, 16)] = jnp.zeros((16,), jnp.float32)
            return carry

        lax.fori_loop(0, ZBR, zf, 0)
        for t in range(4):
            pltpu.sync_copy(zb, out_sh.at[pl.ds(sid * ZB + t * ZBR, ZBR)])
        d1.wait()
        d2.wait()
        d3.wait()
        plsc.subcore_barrier()

        def g_start(k, b):
            pltpu.async_copy(xs_hbm.at[src_v.at[k]], rows[b], gsem[b])

        def g_wait(b):
            pltpu.make_async_copy(xs_hbm.at[src_v.at[0]], rows[b],
                                  gsem[b]).wait()

        def s_start(k, b):
            pltpu.async_copy(rows[b], out_sh.at[dst_v.at[k]], ssem[b],
                             add=True)

        def s_wait(k, b):
            pltpu.make_async_copy(rows[b], out_sh.at[dst_v.at[k]],
                                  ssem[b]).wait()

        def scale(k, b):
            def grp(g, c2):
                base = g * 16
                wv = ew_v[k, pl.ds(base, 16)]
                for l in range(16):
                    bv = jnp.full((16,), wv[l], jnp.float32)
                    e = base + l
                    for q in range(D // 16):
                        sl = pl.ds(q * 16, 16)
                        rows[b][e, sl] = rows[b][e, sl] * bv
                return c2

            lax.fori_loop(0, _CH // 16, grp, 0, unroll=4)

        for b in range(4):
            g_start(b, b)

        def quad(i, carry):
            for b in range(4):
                k = 4 * i + b
                g_wait(b)
                scale(k, b)
                s_start(k, b)
            for b in range(4):
                k = 4 * i + b
                s_wait(k, b)

                @pl.when(k + 4 < rpt)
                def _():
                    g_start(k + 4, b)
            return carry

        lax.fori_loop(0, rpt // 4, quad, 0)
        plsc.subcore_barrier()
        pltpu.sync_copy(out_sh.at[pl.ds(sid * ZB, ZB)],
                        out_hbm.at[cid, pl.ds(sid * ZB, ZB)])

    return pl.kernel(
        body,
        out_type=jax.ShapeDtypeStruct((_NC, NP, D), jnp.float32),
        mesh=mesh,
        scratch_types=[
            pltpu.VMEM((rpt, _CH), jnp.int32),
            pltpu.VMEM((rpt, _CH), jnp.int32),
            pltpu.VMEM((rpt, _CH), jnp.float32),
            pltpu.VMEM((_CH, D), jnp.float32),
            pltpu.VMEM((_CH, D), jnp.float32),
            pltpu.VMEM((_CH, D), jnp.float32),
            pltpu.VMEM((_CH, D), jnp.float32),
            pltpu.VMEM_SHARED((NP, D), jnp.float32),
            pltpu.VMEM((ZBR, D), jnp.float32),
            pltpu.SemaphoreType.DMA,
            pltpu.SemaphoreType.DMA,
            pltpu.SemaphoreType.DMA,
            pltpu.SemaphoreType.DMA,
            pltpu.SemaphoreType.DMA,
            pltpu.SemaphoreType.DMA,
            pltpu.SemaphoreType.DMA,
            pltpu.SemaphoreType.DMA,
            pltpu.SemaphoreType.DMA,
        ],
        compiler_params=pltpu.CompilerParams(use_tc_tiling_on_sc=False),
    )(src2d, dst2d, ew2d, xs)


def _tc_prep(deg_p, xpad, Wc1):
    NPl, D = xpad.shape

    def body(degp_ref, x_ref, w_ref, dinv_ref, xs_ref):
        deg = degp_ref[0:1, :] + degp_ref[1:2, :] + 1.0
        dinv = lax.rsqrt(deg)
        dcol = jnp.transpose(dinv)
        xw = jnp.dot(x_ref[...], w_ref[...], preferred_element_type=jnp.float32)
        dinv_ref[...] = dcol
        xs_ref[...] = dcol * xw

    return pl.pallas_call(
        body,
        out_shape=[jax.ShapeDtypeStruct((NPl, 1), jnp.float32),
                   jax.ShapeDtypeStruct((NPl, D), jnp.float32)],
    )(deg_p, xpad, Wc1)


def _tc_mid(S_p, xs, dinv, bc, Wc):
    NPl, D = xs.shape

    def body(s_ref, xs_ref, dinv_ref, bc_ref, w_ref, out_ref):
        pre = dinv_ref[...] * (s_ref[0] + s_ref[1] + xs_ref[...]) + bc_ref[...]
        h = _selu(pre)
        out_ref[...] = dinv_ref[...] * jnp.dot(
            h, w_ref[...], preferred_element_type=jnp.float32)

    return pl.pallas_call(
        body,
        out_shape=jax.ShapeDtypeStruct((NPl, D), jnp.float32),
    )(S_p, xs, dinv, bc, Wc)


def _tc_final(S_p, xs2, dinv, bc2, W1t, b1t, W2t, b2t, g_col, lb_col,
              block=512):
    D = b1t.shape[0]
    N = b1t.shape[1]
    NPl = xs2.shape[0]
    grid = NPl // block

    def body(s_ref, xs_ref, dinv_ref, bc_ref, w1_ref, b1_ref,
             w2_ref, b2_ref, g_ref, lb_ref, out_ref):
        pre = dinv_ref[...] * (s_ref[0] + s_ref[1] + xs_ref[...]) + bc_ref[...]
        h = _selu(pre)
        ht = jnp.transpose(h)
        t = jnp.sum(w1_ref[...] * ht[:, None, :], axis=0) + b1_ref[...]
        mu = jnp.mean(t, axis=0, keepdims=True)
        var = jnp.mean((t - mu) ** 2, axis=0, keepdims=True)
        t = (t - mu) * lax.rsqrt(var + 1e-5) * g_ref[...] + lb_ref[...]
        out_ref[...] = jnp.sum(w2_ref[...] * t[:, None, :], axis=0) + b2_ref[...]

    return pl.pallas_call(
        body,
        grid=(grid,),
        in_specs=[
            pl.BlockSpec((2, block, D), lambda i: (0, i, 0)),
            pl.BlockSpec((block, D), lambda i: (i, 0)),
            pl.BlockSpec((block, 1), lambda i: (i, 0)),
            pl.BlockSpec((1, D), lambda i: (0, 0)),
            pl.BlockSpec((D, D, block), lambda i: (0, 0, i)),
            pl.BlockSpec((D, block), lambda i: (0, i)),
            pl.BlockSpec((D, D, block), lambda i: (0, 0, i)),
            pl.BlockSpec((D, block), lambda i: (0, i)),
            pl.BlockSpec((D, 1), lambda i: (0, 0)),
            pl.BlockSpec((D, 1), lambda i: (0, 0)),
        ],
        out_specs=pl.BlockSpec((D, block), lambda i: (0, i)),
        out_shape=jax.ShapeDtypeStruct((D, N), jnp.float32),
    )(S_p, xs2, dinv, bc2, W1t, b1t, W2t, b2t, g_col, lb_col)


def kernel(x, edge_index, edge_weight, Wc1, bc1, Wc2, bc2, W1, b1, W2, b2,
           ln_g, ln_b):
    N, D = x.shape
    E = edge_weight.shape[0]
    NP = ((N + 2047) // 2048) * 2048
    if NP == N:
        NP = N + 2048
    EP = ((E + _NW * _CH - 1) // (_NW * _CH)) * (_NW * _CH)
    pad = EP - E

    src = edge_index[0].astype(jnp.int32)
    dst = edge_index[1].astype(jnp.int32)
    ar = jnp.arange(pad, dtype=jnp.int32)
    src_p = jnp.concatenate([src, ar % N]).reshape(EP // _CH, _CH)
    dst_p = jnp.concatenate([dst, N + ar % (NP - N)]).reshape(EP // _CH, _CH)
    ew_p = jnp.concatenate(
        [edge_weight, jnp.zeros((pad,), edge_weight.dtype)]).reshape(EP // _CH, _CH)
    xpad = jnp.concatenate([x, jnp.zeros((NP - N, D), x.dtype)], axis=0)

    deg_p = _sc_deg(dst_p, ew_p, NP).reshape(_NC, NP)
    dinv, xs1 = _tc_prep(deg_p, xpad, Wc1)
    S1 = _sc_msg(src_p, dst_p, ew_p, xs1, NP, D)
    xs2 = _tc_mid(S1, xs1, dinv, bc1.reshape(1, D), Wc2)
    S2 = _sc_msg(src_p, dst_p, ew_p, xs2, NP, D)
    W1t = jnp.transpose(W1, (1, 2, 0))
    W2t = jnp.transpose(W2, (1, 2, 0))
    out_t = _tc_final(S2, xs2, dinv, bc2.reshape(1, D), W1t, jnp.transpose(b1),
                      W2t, jnp.transpose(b2), ln_g.reshape(D, 1),
                      ln_b.reshape(D, 1))
    return jnp.transpose(out_t)


# speedup vs baseline: 1.0104x; 1.0012x over previous
import functools

import jax
import jax.numpy as jnp
from jax import lax
from jax.experimental import pallas as pl
from jax.experimental.pallas import tpu as pltpu
from jax.experimental.pallas import tpu_sc as plsc

_NC = 2
_NS = 16
_NW = _NC * _NS
_CH = 128

_SELU_ALPHA = 1.6732632423543772
_SELU_SCALE = 1.0507009873554805


def _selu(x):
    return _SELU_SCALE * jnp.where(x > 0, x, _SELU_ALPHA * (jnp.exp(x) - 1.0))


def _sc_deg(dst2d, ew2d, NP):
    EPr = dst2d.shape[0]
    rpt = EPr // _NW
    ZB = NP // _NS
    mesh = plsc.VectorSubcoreMesh(core_axis_name="c", subcore_axis_name="s")

    def body(dst_hbm, ew_hbm, out_hbm, dst_v, ew_v, deg_sh, zb, isem, ssem):
        cid = lax.axis_index("c")
        sid = lax.axis_index("s")
        wid = sid * _NC + cid
        row0 = wid * rpt

        d1 = pltpu.async_copy(dst_hbm.at[pl.ds(row0, rpt)], dst_v, isem)
        d2 = pltpu.async_copy(ew_hbm.at[pl.ds(row0, rpt)], ew_v, isem)

        def zf(i, carry):
            zb[pl.ds(i * 16, 16)] = jnp.zeros((16,), jnp.float32)
            return carry

        lax.fori_loop(0, ZB // 16, zf, 0)
        pltpu.sync_copy(zb, deg_sh.at[pl.ds(sid * ZB, ZB)])
        d1.wait()
        d2.wait()
        plsc.subcore_barrier()

        def fire(k, carry):
            pltpu.async_copy(ew_v.at[k], deg_sh.at[dst_v.at[k]], ssem,
                             add=True)
            return carry

        lax.fori_loop(0, rpt, fire, 0)

        def drain(k, carry):
            pltpu.make_async_copy(ew_v.at[0], deg_sh.at[dst_v.at[0]],
                                  ssem).wait()
            return carry

        lax.fori_loop(0, rpt, drain, 0)
        plsc.subcore_barrier()
        pltpu.sync_copy(deg_sh.at[pl.ds(sid * ZB, ZB)],
                        out_hbm.at[pl.ds(cid * NP + sid * ZB, ZB)])

    return pl.kernel(
        body,
        out_type=jax.ShapeDtypeStruct((_NC * NP,), jnp.float32),
        mesh=mesh,
        scratch_types=[
            pltpu.VMEM((rpt, _CH), jnp.int32),
            pltpu.VMEM((rpt, _CH), jnp.float32),
            pltpu.VMEM_SHARED((NP,), jnp.float32),
            pltpu.VMEM((ZB,), jnp.float32),
            pltpu.SemaphoreType.DMA,
            pltpu.SemaphoreType.DMA,
        ],
    )(dst2d, ew2d)


def _sc_msg(src2d, dst2d, ew2d, xs, NP, D):
    EPr = src2d.shape[0]
    rpt = EPr // _NW
    ZB = NP // _NS
    ZBR = ZB // 4
    mesh = plsc.VectorSubcoreMesh(core_axis_name="c", subcore_axis_name="s")

    def body(src_hbm, dst_hbm, ew_hbm, xs_hbm, out_hbm,
             src_v, dst_v, ew_v, rows0, rows1, rows2, rows3, out_sh, zb,
             gsem0, gsem1, gsem2, gsem3, ssem0, ssem1, ssem2, ssem3, isem):
        cid = lax.axis_index("c")
        sid = lax.axis_index("s")
        wid = sid * _NC + cid
        row0 = wid * rpt
        rows = (rows0, rows1, rows2, rows3)
        gsem = (gsem0, gsem1, gsem2, gsem3)
        ssem = (ssem0, ssem1, ssem2, ssem3)

        d1 = pltpu.async_copy(src_hbm.at[pl.ds(row0, rpt)], src_v, isem)
        d2 = pltpu.async_copy(dst_hbm.at[pl.ds(row0, rpt)], dst_v, isem)
        d3 = pltpu.async_copy(ew_hbm.at[pl.ds(row0, rpt)], ew_v, isem)

        def zf(i, carry):
            for g in range(D // 16):
                zb[i, pl.ds(g * 16, 16)] = jnp.zeros((16,), jnp.float32)
            return carry

        lax.fori_loop(0, ZBR, zf, 0)
        for t in range(4):
            pltpu.sync_copy(zb, out_sh.at[pl.ds(sid * ZB + t * ZBR, ZBR)])
        d1.wait()
        d2.wait()
        d3.wait()
        plsc.subcore_barrier()

        def g_start(k, b):
            pltpu.async_copy(xs_hbm.at[src_v.at[k]], rows[b], gsem[b])

        def g_wait(b):
            pltpu.make_async_copy(xs_hbm.at[src_v.at[0]], rows[b],
                                  gsem[b]).wait()

        def s_start(k, b):
            pltpu.async_copy(rows[b], out_sh.at[dst_v.at[k]], ssem[b],
                             add=True)

        def s_wait(k, b):
            pltpu.make_async_copy(rows[b], out_sh.at[dst_v.at[k]],
                                  ssem[b]).wait()

        def scale(k, b):
            def grp(g, c2):
                base = g * 16
                wv = ew_v[k, pl.ds(base, 16)]
                for l in range(16):
                    bv = jnp.full((16,), wv[l], jnp.float32)
                    e = base + l
                    for q in range(D // 16):
                        sl = pl.ds(q * 16, 16)
                        rows[b][e, sl] = rows[b][e, sl] * bv
                return c2

            lax.fori_loop(0, _CH // 16, grp, 0, unroll=2)

        for b in range(4):
            g_start(b, b)

        def quad(i, carry):
            for b in range(4):
                k = 4 * i + b
                g_wait(b)
                scale(k, b)
                s_start(k, b)
            for b in range(4):
                k = 4 * i + b
                s_wait(k, b)

                @pl.when(k + 4 < rpt)
                def _():
                    g_start(k + 4, b)
            return carry

        lax.fori_loop(0, rpt // 4, quad, 0)
        plsc.subcore_barrier()
        pltpu.sync_copy(out_sh.at[pl.ds(sid * ZB, ZB)],
                        out_hbm.at[cid, pl.ds(sid * ZB, ZB)])

    return pl.kernel(
        body,
        out_type=jax.ShapeDtypeStruct((_NC, NP, D), jnp.float32),
        mesh=mesh,
        scratch_types=[
            pltpu.VMEM((rpt, _CH), jnp.int32),
            pltpu.VMEM((rpt, _CH), jnp.int32),
            pltpu.VMEM((rpt, _CH), jnp.float32),
            pltpu.VMEM((_CH, D), jnp.float32),
            pltpu.VMEM((_CH, D), jnp.float32),
            pltpu.VMEM((_CH, D), jnp.float32),
            pltpu.VMEM((_CH, D), jnp.float32),
            pltpu.VMEM_SHARED((NP, D), jnp.float32),
            pltpu.VMEM((ZBR, D), jnp.float32),
            pltpu.SemaphoreType.DMA,
            pltpu.SemaphoreType.DMA,
            pltpu.SemaphoreType.DMA,
            pltpu.SemaphoreType.DMA,
            pltpu.SemaphoreType.DMA,
            pltpu.SemaphoreType.DMA,
            pltpu.SemaphoreType.DMA,
            pltpu.SemaphoreType.DMA,
            pltpu.SemaphoreType.DMA,
        ],
        compiler_params=pltpu.CompilerParams(use_tc_tiling_on_sc=False),
    )(src2d, dst2d, ew2d, xs)


def _tc_prep(deg_p, xpad, Wc1):
    NPl, D = xpad.shape

    def body(degp_ref, x_ref, w_ref, dinv_ref, xs_ref):
        deg = degp_ref[0:1, :] + degp_ref[1:2, :] + 1.0
        dinv = lax.rsqrt(deg)
        dcol = jnp.transpose(dinv)
        xw = jnp.dot(x_ref[...], w_ref[...], preferred_element_type=jnp.float32)
        dinv_ref[...] = dcol
        xs_ref[...] = dcol * xw

    return pl.pallas_call(
        body,
        out_shape=[jax.ShapeDtypeStruct((NPl, 1), jnp.float32),
                   jax.ShapeDtypeStruct((NPl, D), jnp.float32)],
    )(deg_p, xpad, Wc1)


def _tc_mid(S_p, xs, dinv, bc, Wc, block=2560):
    NPl, D = xs.shape
    grid = NPl // block

    def body(s_ref, xs_ref, dinv_ref, bc_ref, w_ref, out_ref):
        pre = dinv_ref[...] * (s_ref[0] + s_ref[1] + xs_ref[...]) + bc_ref[...]
        h = _selu(pre)
        out_ref[...] = dinv_ref[...] * jnp.dot(
            h, w_ref[...], preferred_element_type=jnp.float32)

    return pl.pallas_call(
        body,
        grid=(grid,),
        in_specs=[
            pl.BlockSpec((2, block, D), lambda i: (0, i, 0)),
            pl.BlockSpec((block, D), lambda i: (i, 0)),
            pl.BlockSpec((block, 1), lambda i: (i, 0)),
            pl.BlockSpec((1, D), lambda i: (0, 0)),
            pl.BlockSpec((D, D), lambda i: (0, 0)),
        ],
        out_specs=pl.BlockSpec((block, D), lambda i: (i, 0)),
        out_shape=jax.ShapeDtypeStruct((NPl, D), jnp.float32),
    )(S_p, xs, dinv, bc, Wc)


def _tc_final(S_p, xs2, dinv, bc2, W1t, b1t, W2t, b2t, g_col, lb_col,
              block=512):
    D = b1t.shape[0]
    N = b1t.shape[1]
    NPl = xs2.shape[0]
    grid = NPl // block

    def body(s_ref, xs_ref, dinv_ref, bc_ref, w1_ref, b1_ref,
             w2_ref, b2_ref, g_ref, lb_ref, out_ref):
        pre = dinv_ref[...] * (s_ref[0] + s_ref[1] + xs_ref[...]) + bc_ref[...]
        h = _selu(pre)
        ht = jnp.transpose(h)
        t = jnp.sum(w1_ref[...] * ht[:, None, :], axis=0) + b1_ref[...]
        mu = jnp.mean(t, axis=0, keepdims=True)
        var = jnp.mean((t - mu) ** 2, axis=0, keepdims=True)
        t = (t - mu) * lax.rsqrt(var + 1e-5) * g_ref[...] + lb_ref[...]
        out_ref[...] = jnp.sum(w2_ref[...] * t[:, None, :], axis=0) + b2_ref[...]

    return pl.pallas_call(
        body,
        grid=(grid,),
        in_specs=[
            pl.BlockSpec((2, block, D), lambda i: (0, i, 0)),
            pl.BlockSpec((block, D), lambda i: (i, 0)),
            pl.BlockSpec((block, 1), lambda i: (i, 0)),
            pl.BlockSpec((1, D), lambda i: (0, 0)),
            pl.BlockSpec((D, D, block), lambda i: (0, 0, i)),
            pl.BlockSpec((D, block), lambda i: (0, i)),
            pl.BlockSpec((D, D, block), lambda i: (0, 0, i)),
            pl.BlockSpec((D, block), lambda i: (0, i)),
            pl.BlockSpec((D, 1), lambda i: (0, 0)),
            pl.BlockSpec((D, 1), lambda i: (0, 0)),
        ],
        out_specs=pl.BlockSpec((D, block), lambda i: (0, i)),
        out_shape=jax.ShapeDtypeStruct((D, N), jnp.float32),
    )(S_p, xs2, dinv, bc2, W1t, b1t, W2t, b2t, g_col, lb_col)


def kernel(x, edge_index, edge_weight, Wc1, bc1, Wc2, bc2, W1, b1, W2, b2,
           ln_g, ln_b):
    N, D = x.shape
    E = edge_weight.shape[0]
    NP = ((N + 2047) // 2048) * 2048
    if NP == N:
        NP = N + 2048
    EP = ((E + _NW * _CH - 1) // (_NW * _CH)) * (_NW * _CH)
    pad = EP - E

    src = edge_index[0].astype(jnp.int32)
    dst = edge_index[1].astype(jnp.int32)
    ar = jnp.arange(pad, dtype=jnp.int32)
    src_p = jnp.concatenate([src, ar % N]).reshape(EP // _CH, _CH)
    dst_p = jnp.concatenate([dst, N + ar % (NP - N)]).reshape(EP // _CH, _CH)
    ew_p = jnp.concatenate(
        [edge_weight, jnp.zeros((pad,), edge_weight.dtype)]).reshape(EP // _CH, _CH)
    xpad = jnp.concatenate([x, jnp.zeros((NP - N, D), x.dtype)], axis=0)

    deg_p = _sc_deg(dst_p, ew_p, NP).reshape(_NC, NP)
    dinv, xs1 = _tc_prep(deg_p, xpad, Wc1)
    S1 = _sc_msg(src_p, dst_p, ew_p, xs1, NP, D)
    xs2 = _tc_mid(S1, xs1, dinv, bc1.reshape(1, D), Wc2)
    S2 = _sc_msg(src_p, dst_p, ew_p, xs2, NP, D)
    W1t = jnp.transpose(W1, (1, 2, 0))
    W2t = jnp.transpose(W2, (1, 2, 0))
    out_t = _tc_final(S2, xs2, dinv, bc2.reshape(1, D), W1t, jnp.transpose(b1),
                      W2t, jnp.transpose(b2), ln_g.reshape(D, 1),
                      ln_b.reshape(D, 1))
    return jnp.transpose(out_t)


# speedup vs baseline: 1.0281x; 1.0176x over previous
import functools

import jax
import jax.numpy as jnp
from jax import lax
from jax.experimental import pallas as pl
from jax.experimental.pallas import tpu as pltpu
from jax.experimental.pallas import tpu_sc as plsc

_NC = 2
_NS = 16
_NW = _NC * _NS
_CH = 128

_SELU_ALPHA = 1.6732632423543772
_SELU_SCALE = 1.0507009873554805


def _selu(x):
    return _SELU_SCALE * jnp.where(x > 0, x, _SELU_ALPHA * (jnp.exp(x) - 1.0))


def _sc_deg(dst2d, ew2d, NP):
    EPr = dst2d.shape[0]
    rpt = EPr // _NW
    ZB = NP // _NS
    mesh = plsc.VectorSubcoreMesh(core_axis_name="c", subcore_axis_name="s")

    def body(dst_hbm, ew_hbm, out_hbm, dst_v, ew_v, deg_sh, zb, isem, ssem):
        cid = lax.axis_index("c")
        sid = lax.axis_index("s")
        wid = sid * _NC + cid
        row0 = wid * rpt

        d1 = pltpu.async_copy(dst_hbm.at[pl.ds(row0, rpt)], dst_v, isem)
        d2 = pltpu.async_copy(ew_hbm.at[pl.ds(row0, rpt)], ew_v, isem)

        def zf(i, carry):
            zb[pl.ds(i * 16, 16)] = jnp.zeros((16,), jnp.float32)
            return carry

        lax.fori_loop(0, ZB // 16, zf, 0)
        pltpu.sync_copy(zb, deg_sh.at[pl.ds(sid * ZB, ZB)])
        d1.wait()
        d2.wait()
        plsc.subcore_barrier()

        def fire(k, carry):
            pltpu.async_copy(ew_v.at[k], deg_sh.at[dst_v.at[k]], ssem,
                             add=True)
            return carry

        lax.fori_loop(0, rpt, fire, 0)

        def drain(k, carry):
            pltpu.make_async_copy(ew_v.at[0], deg_sh.at[dst_v.at[0]],
                                  ssem).wait()
            return carry

        lax.fori_loop(0, rpt, drain, 0)
        plsc.subcore_barrier()
        pltpu.sync_copy(deg_sh.at[pl.ds(sid * ZB, ZB)],
                        out_hbm.at[pl.ds(cid * NP + sid * ZB, ZB)])

    return pl.kernel(
        body,
        out_type=jax.ShapeDtypeStruct((_NC * NP,), jnp.float32),
        mesh=mesh,
        scratch_types=[
            pltpu.VMEM((rpt, _CH), jnp.int32),
            pltpu.VMEM((rpt, _CH), jnp.float32),
            pltpu.VMEM_SHARED((NP,), jnp.float32),
            pltpu.VMEM((ZB,), jnp.float32),
            pltpu.SemaphoreType.DMA,
            pltpu.SemaphoreType.DMA,
        ],
    )(dst2d, ew2d)


def _sc_msg(src2d, dst2d, ew2d, xs, NP, D):
    EPr = src2d.shape[0]
    rpt = EPr // _NW
    ZB = NP // _NS
    ZBR = ZB // 4
    mesh = plsc.VectorSubcoreMesh(core_axis_name="c", subcore_axis_name="s")

    def body(src_hbm, dst_hbm, ew_hbm, xs_hbm, out_hbm,
             src_v, dst_v, ew_v, rows0, rows1, rows2, rows3, out_sh, zb,
             gsem0, gsem1, gsem2, gsem3, ssem0, ssem1, ssem2, ssem3, isem):
        cid = lax.axis_index("c")
        sid = lax.axis_index("s")
        wid = sid * _NC + cid
        row0 = wid * rpt
        rows = (rows0, rows1, rows2, rows3)
        gsem = (gsem0, gsem1, gsem2, gsem3)
        ssem = (ssem0, ssem1, ssem2, ssem3)

        d1 = pltpu.async_copy(src_hbm.at[pl.ds(row0, rpt)], src_v, isem)
        d2 = pltpu.async_copy(dst_hbm.at[pl.ds(row0, rpt)], dst_v, isem)
        d3 = pltpu.async_copy(ew_hbm.at[pl.ds(row0, rpt)], ew_v, isem)

        def zf(i, carry):
            for g in range(D // 16):
                zb[i, pl.ds(g * 16, 16)] = jnp.zeros((16,), jnp.float32)
            return carry

        lax.fori_loop(0, ZBR, zf, 0)
        for t in range(4):
            pltpu.sync_copy(zb, out_sh.at[pl.ds(sid * ZB + t * ZBR, ZBR)])
        d1.wait()
        d2.wait()
        d3.wait()
        plsc.subcore_barrier()

        def g_start(k, b):
            pltpu.async_copy(xs_hbm.at[src_v.at[k]], rows[b], gsem[b])

        def g_wait(b):
            pltpu.make_async_copy(xs_hbm.at[src_v.at[0]], rows[b],
                                  gsem[b]).wait()

        def s_start(k, b):
            pltpu.async_copy(rows[b], out_sh.at[dst_v.at[k]], ssem[b],
                             add=True)

        def s_wait(k, b):
            pltpu.make_async_copy(rows[b], out_sh.at[dst_v.at[k]],
                                  ssem[b]).wait()

        def scale(k, b):
            def grp(g, c2):
                base = g * 16
                wv = ew_v[k, pl.ds(base, 16)]
                for l in range(16):
                    bv = jnp.full((16,), wv[l], jnp.float32)
                    e = base + l
                    for q in range(D // 16):
                        sl = pl.ds(q * 16, 16)
                        rows[b][e, sl] = rows[b][e, sl] * bv
                return c2

            lax.fori_loop(0, _CH // 16, grp, 0, unroll=2)

        for b in range(4):
            g_start(b, b)

        def quad(i, carry):
            for b in range(4):
                k = 4 * i + b
                g_wait(b)
                scale(k, b)
                s_start(k, b)
            for b in range(4):
                k = 4 * i + b
                s_wait(k, b)

                @pl.when(k + 4 < rpt)
                def _():
                    g_start(k + 4, b)
            return carry

        lax.fori_loop(0, rpt // 4, quad, 0)
        plsc.subcore_barrier()
        pltpu.sync_copy(out_sh.at[pl.ds(sid * ZB, ZB)],
                        out_hbm.at[cid, pl.ds(sid * ZB, ZB)])

    return pl.kernel(
        body,
        out_type=jax.ShapeDtypeStruct((_NC, NP, D), jnp.float32),
        mesh=mesh,
        scratch_types=[
            pltpu.VMEM((rpt, _CH), jnp.int32),
            pltpu.VMEM((rpt, _CH), jnp.int32),
            pltpu.VMEM((rpt, _CH), jnp.float32),
            pltpu.VMEM((_CH, D), jnp.float32),
            pltpu.VMEM((_CH, D), jnp.float32),
            pltpu.VMEM((_CH, D), jnp.float32),
            pltpu.VMEM((_CH, D), jnp.float32),
            pltpu.VMEM_SHARED((NP, D), jnp.float32),
            pltpu.VMEM((ZBR, D), jnp.float32),
            pltpu.SemaphoreType.DMA,
            pltpu.SemaphoreType.DMA,
            pltpu.SemaphoreType.DMA,
            pltpu.SemaphoreType.DMA,
            pltpu.SemaphoreType.DMA,
            pltpu.SemaphoreType.DMA,
            pltpu.SemaphoreType.DMA,
            pltpu.SemaphoreType.DMA,
            pltpu.SemaphoreType.DMA,
        ],
        compiler_params=pltpu.CompilerParams(use_tc_tiling_on_sc=False),
    )(src2d, dst2d, ew2d, xs)


def _tc_prep(deg_p, xpad, Wc1):
    NPl, D = xpad.shape

    def body(degp_ref, x_ref, w_ref, dinv_ref, xs_ref):
        deg = degp_ref[0:1, :] + degp_ref[1:2, :] + 1.0
        dinv = lax.rsqrt(deg)
        dcol = jnp.transpose(dinv)
        xw = jnp.dot(x_ref[...], w_ref[...], preferred_element_type=jnp.float32)
        dinv_ref[...] = dcol
        xs_ref[...] = dcol * xw

    return pl.pallas_call(
        body,
        out_shape=[jax.ShapeDtypeStruct((NPl, 1), jnp.float32),
                   jax.ShapeDtypeStruct((NPl, D), jnp.float32)],
    )(deg_p, xpad, Wc1)


def _tc_mid(S_p, xs, dinv, bc, Wc, block=2560):
    NPl, D = xs.shape
    grid = NPl // block

    def body(s_ref, xs_ref, dinv_ref, bc_ref, w_ref, out_ref):
        pre = dinv_ref[...] * (s_ref[0] + s_ref[1] + xs_ref[...]) + bc_ref[...]
        h = _selu(pre)
        out_ref[...] = dinv_ref[...] * jnp.dot(
            h, w_ref[...], preferred_element_type=jnp.float32)

    return pl.pallas_call(
        body,
        grid=(grid,),
        in_specs=[
            pl.BlockSpec((2, block, D), lambda i: (0, i, 0)),
            pl.BlockSpec((block, D), lambda i: (i, 0)),
            pl.BlockSpec((block, 1), lambda i: (i, 0)),
            pl.BlockSpec((1, D), lambda i: (0, 0)),
            pl.BlockSpec((D, D), lambda i: (0, 0)),
        ],
        out_specs=pl.BlockSpec((block, D), lambda i: (i, 0)),
        out_shape=jax.ShapeDtypeStruct((NPl, D), jnp.float32),
    )(S_p, xs, dinv, bc, Wc)


def _tc_final(S_p, xs2, dinv, bc2, W1t, b1t, W2t, b2t, g_col, lb_col,
              block=256):
    D = b1t.shape[0]
    N = b1t.shape[1]
    NPl = xs2.shape[0]
    grid = NPl // block

    def body(s_ref, xs_ref, dinv_ref, bc_ref, w1_ref, b1_ref,
             w2_ref, b2_ref, g_ref, lb_ref, out_ref):
        pre = dinv_ref[...] * (s_ref[0] + s_ref[1] + xs_ref[...]) + bc_ref[...]
        h = _selu(pre)
        ht = jnp.transpose(h)
        t = jnp.sum(w1_ref[...] * ht[:, None, :], axis=0) + b1_ref[...]
        mu = jnp.mean(t, axis=0, keepdims=True)
        var = jnp.mean((t - mu) ** 2, axis=0, keepdims=True)
        t = (t - mu) * lax.rsqrt(var + 1e-5) * g_ref[...] + lb_ref[...]
        out_ref[...] = jnp.sum(w2_ref[...] * t[:, None, :], axis=0) + b2_ref[...]

    return pl.pallas_call(
        body,
        grid=(grid,),
        in_specs=[
            pl.BlockSpec((2, block, D), lambda i: (0, i, 0)),
            pl.BlockSpec((block, D), lambda i: (i, 0)),
            pl.BlockSpec((block, 1), lambda i: (i, 0)),
            pl.BlockSpec((1, D), lambda i: (0, 0)),
            pl.BlockSpec((D, D, block), lambda i: (0, 0, i)),
            pl.BlockSpec((D, block), lambda i: (0, i)),
            pl.BlockSpec((D, D, block), lambda i: (0, 0, i)),
            pl.BlockSpec((D, block), lambda i: (0, i)),
            pl.BlockSpec((D, 1), lambda i: (0, 0)),
            pl.BlockSpec((D, 1), lambda i: (0, 0)),
        ],
        out_specs=pl.BlockSpec((D, block), lambda i: (0, i)),
        out_shape=jax.ShapeDtypeStruct((D, N), jnp.float32),
    )(S_p, xs2, dinv, bc2, W1t, b1t, W2t, b2t, g_col, lb_col)


def kernel(x, edge_index, edge_weight, Wc1, bc1, Wc2, bc2, W1, b1, W2, b2,
           ln_g, ln_b):
    N, D = x.shape
    E = edge_weight.shape[0]
    NP = ((N + 2047) // 2048) * 2048
    if NP == N:
        NP = N + 2048
    EP = ((E + _NW * _CH - 1) // (_NW * _CH)) * (_NW * _CH)
    pad = EP - E

    src = edge_index[0].astype(jnp.int32)
    dst = edge_index[1].astype(jnp.int32)
    ar = jnp.arange(pad, dtype=jnp.int32)
    src_p = jnp.concatenate([src, ar % N]).reshape(EP // _CH, _CH)
    dst_p = jnp.concatenate([dst, N + ar % (NP - N)]).reshape(EP // _CH, _CH)
    ew_p = jnp.concatenate(
        [edge_weight, jnp.zeros((pad,), edge_weight.dtype)]).reshape(EP // _CH, _CH)
    xpad = jnp.concatenate([x, jnp.zeros((NP - N, D), x.dtype)], axis=0)

    deg_p = _sc_deg(dst_p, ew_p, NP).reshape(_NC, NP)
    dinv, xs1 = _tc_prep(deg_p, xpad, Wc1)
    S1 = _sc_msg(src_p, dst_p, ew_p, xs1, NP, D)
    xs2 = _tc_mid(S1, xs1, dinv, bc1.reshape(1, D), Wc2)
    S2 = _sc_msg(src_p, dst_p, ew_p, xs2, NP, D)
    W1t = jnp.transpose(W1, (1, 2, 0))
    W2t = jnp.transpose(W2, (1, 2, 0))
    out_t = _tc_final(S2, xs2, dinv, bc2.reshape(1, D), W1t, jnp.transpose(b1),
                      W2t, jnp.transpose(b2), ln_g.reshape(D, 1),
                      ln_b.reshape(D, 1))
    return jnp.transpose(out_t)
